# Initial kernel scaffold; baseline (speedup 1.0000x reference)
import jax
import jax.numpy as jnp
from jax.experimental import pallas as pl


def kernel(yi, Z, Rij, idx_i, idx_j, idx_m):
    raise NotImplementedError("write your pallas kernel here")


# speedup vs baseline: 270.7576x; 270.7576x over previous
import functools

import jax
import jax.numpy as jnp
from jax import lax
from jax.experimental import pallas as pl
from jax.experimental.pallas import tpu as pltpu
from jax.experimental.pallas import tpu_sc as plsc

KE = 14.399645351950548
KEHALF = KE / 2.0
CUTOFF = 10.0
CUTON = 2.5
EPS = 1e-9
A0 = 0.5291772105638411
INV_ADIV = 1.0 / (0.8854 * A0)

NA = 100000
NE = 6400000
NM = 64
NA_PAD = 102400
ROWS = NA_PAD // 128
NC, NS, L = 2, 16, 16
NW = NC * NS
CH = 2048
KROW = CH // 128
NCHUNK = NE // CH
TMAX = -(-NCHUNK // NW)
APT = NA_PAD // NS


def _atom_body(charge_ref, zf_ref, idxm_ref, out_ref):
    charge = charge_ref[...]
    zf = zf_ref[...]
    idxm = idxm_ref[...]
    vals = []
    for m in range(NM):
        mask = idxm == m
        qs = jnp.sum(jnp.where(mask, charge, 0.0))
        cnt = jnp.sum(jnp.where(mask, 1.0, 0.0))
        vals.append(-qs / jnp.maximum(cnt, 1.0))
    adj = jnp.zeros_like(charge)
    for m in range(NM):
        adj = jnp.where(idxm == m, vals[m], adj)
    qa = charge + adj
    alpha = 0.4 * zf + 1.0
    c6a = alpha * alpha * jnp.exp(-0.5 * qa)
    p023 = jnp.where(zf > 0.0, jnp.exp(0.23 * jnp.log(jnp.maximum(zf, 1e-30))), 0.0)
    out_ref[0] = qa
    out_ref[1] = c6a
    out_ref[2] = zf
    out_ref[3] = p023


def _stage1(charge2, zf2, idxm2):
    return pl.pallas_call(
        _atom_body,
        out_shape=jax.ShapeDtypeStruct((4, ROWS, 128), jnp.float32),
    )(charge2, zf2, idxm2)


def _rsqrt(x):
    i = plsc.bitcast(x, jnp.int32)
    i = 0x5F3759DF - lax.shift_right_logical(i, 1)
    y = plsc.bitcast(i, jnp.float32)
    for _ in range(3):
        y = y * (1.5 - 0.5 * x * y * y)
    return y


def _sc_body(qa_h, c6_h, zf_h, p0_h, rij_h, idxi_h, idxj_h, zero_h, out_hbm,
             qa_s, c6_s, zf_s, p0_s, acc_s,
             idxi_v, idxj_v, rx_v, ry_v, rz_v,
             qi_v, qj_v, c6i_v, c6j_v, zi_v, zj_v, pi_v, pj_v,
             epair_v, zbuf, semL, semG):
    c = lax.axis_index("c")
    s = lax.axis_index("s")
    w = s * NC + c

    sl = pl.ds(s * APT, APT)
    for src_h, dst_s in ((qa_h, qa_s), (c6_h, c6_s), (zf_h, zf_s),
                         (p0_h, p0_s), (zero_h, acc_s)):
        pltpu.sync_copy(src_h.at[sl], zbuf)
        pltpu.sync_copy(zbuf, dst_s.at[sl])
    plsc.subcore_barrier()

    def chunk_body(t, carry):
        cid = w + NW * t

        @pl.when(cid < NCHUNK)
        def _():
            dsc = [pltpu.async_copy(idxi_h.at[cid], idxi_v, semL),
                   pltpu.async_copy(idxj_h.at[cid], idxj_v, semL),
                   pltpu.async_copy(rij_h.at[0, cid], rx_v, semL),
                   pltpu.async_copy(rij_h.at[1, cid], ry_v, semL),
                   pltpu.async_copy(rij_h.at[2, cid], rz_v, semL)]
            for d in dsc:
                d.wait()

            gds = []
            for k in range(KROW):
                ik = idxi_v.at[k]
                jk = idxj_v.at[k]
                ds128 = pl.ds(k * 128, 128)
                for tab, dst in ((qa_s, qi_v), (c6_s, c6i_v),
                                 (zf_s, zi_v), (p0_s, pi_v)):
                    gds.append(pltpu.async_copy(tab.at[ik], dst.at[ds128], semG))
                for tab, dst in ((qa_s, qj_v), (c6_s, c6j_v),
                                 (zf_s, zj_v), (p0_s, pj_v)):
                    gds.append(pltpu.async_copy(tab.at[jk], dst.at[ds128], semG))
            for g in gds:
                g.wait()

            def vbody(j, vc):
                b = pl.ds(j * L, L)
                xv = rx_v[b]
                yv = ry_v[b]
                zv = rz_v[b]
                qi = qi_v[b]
                qj = qj_v[b]
                c6i = c6i_v[b]
                c6j = c6j_v[b]
                zi = zi_v[b]
                zj = zj_v[b]
                pi_ = pi_v[b]
                pj_ = pj_v[b]

                d2v = xv * xv + yv * yv + zv * zv
                rinv = _rsqrt(jnp.maximum(d2v, 1e-18))
                d = d2v * rinv
                dsafe = jnp.maximum(d, EPS)
                inv_d = 1.0 / dsafe
                incut = d < CUTOFF
                xc = d * (1.0 / CUTOFF)
                xc3 = xc * xc * xc
                fcut = jnp.where(
                    incut, 1.0 - xc3 * ((6.0 * xc - 15.0) * xc + 10.0), 0.0)
                x1 = jnp.clip((d - CUTON) * 0.2, 0.0, 1.0)
                sw = x1 * x1 * x1 * (x1 * (6.0 * x1 - 15.0) + 10.0)
                damped = _rsqrt(d2v + 1.0)
                coul = (1.0 - sw) * damped + sw * inv_d
                coul = jnp.where(incut, coul + d * 0.01 - 0.2, 0.0)
                ec = KEHALF * qi * qj * coul

                p = jnp.maximum(c6i * c6j, EPS)
                c6ij = p * _rsqrt(p)
                ds2 = dsafe * dsafe
                d6 = ds2 * ds2 * ds2
                r0 = 2.0 + 0.1 * (zi + zj)
                r02 = r0 * r0
                r06 = r02 * r02 * r02
                ea = (-0.5 * fcut) * c6ij / (d6 + r06)

                xs = dsafe * ((pi_ + pj_ + EPS) * INV_ADIV)
                phi = (0.18175 * jnp.exp(-3.19980 * xs)
                       + 0.50986 * jnp.exp(-0.94229 * xs)
                       + 0.28022 * jnp.exp(-0.40290 * xs)
                       + 0.02817 * jnp.exp(-0.20162 * xs))
                ez = KEHALF * zi * zj * inv_d * phi * fcut

                epair_v[b] = ec + ea + ez
                return vc

            lax.fori_loop(0, CH // L, vbody, 0)

            for k in range(KROW):
                pltpu.sync_copy(epair_v.at[pl.ds(k * 128, 128)],
                                acc_s.at[idxi_v.at[k]], add=True)

        return carry

    lax.fori_loop(0, TMAX, chunk_body, 0)

    plsc.subcore_barrier()
    pltpu.sync_copy(acc_s.at[pl.ds(s * APT, APT)], zbuf)
    pltpu.sync_copy(zbuf, out_hbm.at[c, pl.ds(s * APT, APT)])


_sc_call = functools.partial(
    pl.kernel,
    out_type=jax.ShapeDtypeStruct((NC, NA_PAD), jnp.float32),
    mesh=plsc.VectorSubcoreMesh(
        core_axis_name="c", subcore_axis_name="s",
        num_cores=NC, num_subcores=NS),
    compiler_params=pltpu.CompilerParams(needs_layout_passes=False),
    scratch_types=(
        [pltpu.VMEM_SHARED((NA_PAD,), jnp.float32) for _ in range(5)]
        + [pltpu.VMEM((KROW, 128), jnp.int32) for _ in range(2)]
        + [pltpu.VMEM((CH,), jnp.float32) for _ in range(12)]
        + [pltpu.VMEM((APT,), jnp.float32),
           pltpu.SemaphoreType.DMA,
           pltpu.SemaphoreType.DMA]
    ),
)(_sc_body)


def kernel(yi, Z, Rij, idx_i, idx_j, idx_m):
    charge = yi[:, 1]
    zf = Z.astype(jnp.float32)
    pad = NA_PAD - NA
    charge2 = jnp.pad(charge, (0, pad)).reshape(ROWS, 128)
    zf2 = jnp.pad(zf, (0, pad)).reshape(ROWS, 128)
    idxm2 = jnp.pad(idx_m, (0, pad), constant_values=NM).reshape(ROWS, 128)
    tab = _stage1(charge2, zf2, idxm2).reshape(4, NA_PAD)

    rijT = Rij.T.reshape(3, NCHUNK, CH)
    idxi3 = idx_i.reshape(NCHUNK, KROW, 128)
    idxj3 = idx_j.reshape(NCHUNK, KROW, 128)
    zero = jnp.zeros((NA_PAD,), jnp.float32)
    partials = _sc_call(tab[0], tab[1], tab[2], tab[3],
                        rijT, idxi3, idxj3, zero)
    return (yi[:, 0] + partials[0, :NA] + partials[1, :NA])[:, None]


# speedup vs baseline: 327.8414x; 1.2108x over previous
import functools

import jax
import jax.numpy as jnp
from jax import lax
from jax.experimental import pallas as pl
from jax.experimental.pallas import tpu as pltpu
from jax.experimental.pallas import tpu_sc as plsc

KE = 14.399645351950548
KEHALF = KE / 2.0
CUTOFF = 10.0
CUTON = 2.5
EPS = 1e-9
A0 = 0.5291772105638411
INV_ADIV = 1.0 / (0.8854 * A0)

NA = 100000
NE = 6400000
NM = 64
NA_PAD = 102400
ROWS = NA_PAD // 128
NC, NS, L = 2, 16, 16
NW = NC * NS
CH = 2048
KROW = CH // 128
NCHUNK = NE // CH
TMAX = -(-NCHUNK // NW)
APT = NA_PAD // NS


def _atom_body(charge_ref, idxm_ref, out_ref):
    charge = charge_ref[...]
    idxm = idxm_ref[...]
    vals = []
    for m in range(NM):
        mask = idxm == m
        qs = jnp.sum(jnp.where(mask, charge, 0.0))
        cnt = jnp.sum(jnp.where(mask, 1.0, 0.0))
        vals.append(-qs / jnp.maximum(cnt, 1.0))
    adj = jnp.zeros_like(charge)
    for m in range(NM):
        adj = jnp.where(idxm == m, vals[m], adj)
    out_ref[...] = charge + adj


def _stage1(charge2, idxm2):
    return pl.pallas_call(
        _atom_body,
        out_shape=jax.ShapeDtypeStruct((ROWS, 128), jnp.float32),
    )(charge2, idxm2)


def _rsqrt(x):
    i = plsc.bitcast(x, jnp.int32)
    i = 0x5F3759DF - lax.shift_right_logical(i, 1)
    y = plsc.bitcast(i, jnp.float32)
    for _ in range(3):
        y = y * (1.5 - 0.5 * x * y * y)
    return y


def _sc_body(qa_h, z_h, p0_h, rij_h, idxi_h, idxj_h, zero_h, out_hbm,
             qa_s, z_s, acc_s,
             idxi_v, idxj_v, rx_v, ry_v, rz_v,
             qi_v, qj_v, zi_v, zj_v, p0_v,
             epair_v, zbuf, zbufi, semL, semG):
    c = lax.axis_index("c")
    s = lax.axis_index("s")
    w = s * NC + c

    sl = pl.ds(s * APT, APT)
    for src_h, dst_s in ((qa_h, qa_s), (zero_h, acc_s)):
        pltpu.sync_copy(src_h.at[sl], zbuf)
        pltpu.sync_copy(zbuf, dst_s.at[sl])
    pltpu.sync_copy(z_h.at[sl], zbufi)
    pltpu.sync_copy(zbufi, z_s.at[sl])
    pltpu.sync_copy(p0_h, p0_v)
    plsc.subcore_barrier()

    def chunk_body(t, carry):
        cid = w + NW * t

        @pl.when(cid < NCHUNK)
        def _():
            dsc = [pltpu.async_copy(idxi_h.at[cid], idxi_v, semL),
                   pltpu.async_copy(idxj_h.at[cid], idxj_v, semL),
                   pltpu.async_copy(rij_h.at[0, cid], rx_v, semL),
                   pltpu.async_copy(rij_h.at[1, cid], ry_v, semL),
                   pltpu.async_copy(rij_h.at[2, cid], rz_v, semL)]
            for d in dsc:
                d.wait()

            gds = []
            for k in range(KROW):
                ik = idxi_v.at[k]
                jk = idxj_v.at[k]
                ds128 = pl.ds(k * 128, 128)
                gds.append(pltpu.async_copy(qa_s.at[ik], qi_v.at[ds128], semG))
                gds.append(pltpu.async_copy(qa_s.at[jk], qj_v.at[ds128], semG))
                gds.append(pltpu.async_copy(z_s.at[ik], zi_v.at[ds128], semG))
                gds.append(pltpu.async_copy(z_s.at[jk], zj_v.at[ds128], semG))
            for g in gds:
                g.wait()

            def vbody(j, vc):
                b = pl.ds(j * L, L)
                xv = rx_v[b]
                yv = ry_v[b]
                zv = rz_v[b]
                qi = qi_v[b]
                qj = qj_v[b]
                zi32 = zi_v[b]
                zj32 = zj_v[b]
                zi = zi32.astype(jnp.float32)
                zj = zj32.astype(jnp.float32)
                pi_ = plsc.load_gather(p0_v, [zi32])
                pj_ = plsc.load_gather(p0_v, [zj32])

                d2v = xv * xv + yv * yv + zv * zv
                rinv = _rsqrt(jnp.maximum(d2v, 1e-18))
                d = d2v * rinv
                dsafe = jnp.maximum(d, EPS)
                inv_d = 1.0 / dsafe
                incut = d < CUTOFF
                xc = d * (1.0 / CUTOFF)
                xc3 = xc * xc * xc
                fcut = jnp.where(
                    incut, 1.0 - xc3 * ((6.0 * xc - 15.0) * xc + 10.0), 0.0)
                x1 = jnp.clip((d - CUTON) * 0.2, 0.0, 1.0)
                sw = x1 * x1 * x1 * (x1 * (6.0 * x1 - 15.0) + 10.0)
                damped = _rsqrt(d2v + 1.0)
                coul = (1.0 - sw) * damped + sw * inv_d
                coul = jnp.where(incut, coul + d * 0.01 - 0.2, 0.0)
                ec = KEHALF * qi * qj * coul

                c6ij = ((0.4 * zi + 1.0) * (0.4 * zj + 1.0)
                        * jnp.exp(-0.25 * (qi + qj)))
                ds2 = dsafe * dsafe
                d6 = ds2 * ds2 * ds2
                r0 = 2.0 + 0.1 * (zi + zj)
                r02 = r0 * r0
                r06 = r02 * r02 * r02
                ea = (-0.5 * fcut) * c6ij / (d6 + r06)

                xs = dsafe * ((pi_ + pj_ + EPS) * INV_ADIV)
                phi = (0.18175 * jnp.exp(-3.19980 * xs)
                       + 0.50986 * jnp.exp(-0.94229 * xs)
                       + 0.28022 * jnp.exp(-0.40290 * xs)
                       + 0.02817 * jnp.exp(-0.20162 * xs))
                ez = KEHALF * zi * zj * inv_d * phi * fcut

                epair_v[b] = ec + ea + ez
                return vc

            lax.fori_loop(0, CH // L, vbody, 0)

            for k in range(KROW):
                pltpu.sync_copy(epair_v.at[pl.ds(k * 128, 128)],
                                acc_s.at[idxi_v.at[k]], add=True)

        return carry

    lax.fori_loop(0, TMAX, chunk_body, 0)

    plsc.subcore_barrier()
    pltpu.sync_copy(acc_s.at[pl.ds(s * APT, APT)], zbuf)
    pltpu.sync_copy(zbuf, out_hbm.at[c, pl.ds(s * APT, APT)])


_sc_call = functools.partial(
    pl.kernel,
    out_type=jax.ShapeDtypeStruct((NC, NA_PAD), jnp.float32),
    mesh=plsc.VectorSubcoreMesh(
        core_axis_name="c", subcore_axis_name="s",
        num_cores=NC, num_subcores=NS),
    compiler_params=pltpu.CompilerParams(needs_layout_passes=False),
    scratch_types=(
        [pltpu.VMEM_SHARED((NA_PAD,), jnp.float32),
         pltpu.VMEM_SHARED((NA_PAD,), jnp.int32),
         pltpu.VMEM_SHARED((NA_PAD,), jnp.float32)]
        + [pltpu.VMEM((KROW, 128), jnp.int32) for _ in range(2)]
        + [pltpu.VMEM((CH,), jnp.float32) for _ in range(5)]
        + [pltpu.VMEM((CH,), jnp.int32) for _ in range(2)]
        + [pltpu.VMEM((128,), jnp.float32),
           pltpu.VMEM((CH,), jnp.float32),
           pltpu.VMEM((APT,), jnp.float32),
           pltpu.VMEM((APT,), jnp.int32),
           pltpu.SemaphoreType.DMA,
           pltpu.SemaphoreType.DMA]
    ),
)(_sc_body)


def kernel(yi, Z, Rij, idx_i, idx_j, idx_m):
    charge = yi[:, 1]
    pad = NA_PAD - NA
    charge2 = jnp.pad(charge, (0, pad)).reshape(ROWS, 128)
    idxm2 = jnp.pad(idx_m, (0, pad), constant_values=NM).reshape(ROWS, 128)
    qa_h = _stage1(charge2, idxm2).reshape(NA_PAD)
    z_h = jnp.pad(Z, (0, pad))
    p0tab = jnp.arange(128, dtype=jnp.float32) ** 0.23

    rijT = Rij.T.reshape(3, NCHUNK, CH)
    idxi3 = idx_i.reshape(NCHUNK, KROW, 128)
    idxj3 = idx_j.reshape(NCHUNK, KROW, 128)
    zero = jnp.zeros((NA_PAD,), jnp.float32)
    partials = _sc_call(qa_h, z_h, p0tab, rijT, idxi3, idxj3, zero)
    return (yi[:, 0] + partials[0, :NA] + partials[1, :NA])[:, None]


# speedup vs baseline: 345.2432x; 1.0531x over previous
import functools

import jax
import jax.numpy as jnp
from jax import lax
from jax.experimental import pallas as pl
from jax.experimental.pallas import tpu as pltpu
from jax.experimental.pallas import tpu_sc as plsc

KE = 14.399645351950548
KEHALF = KE / 2.0
CUTOFF = 10.0
CUTON = 2.5
EPS = 1e-9
A0 = 0.5291772105638411
INV_ADIV = 1.0 / (0.8854 * A0)

NA = 100000
NE = 6400000
NM = 64
NA_PAD = 102400
ROWS = NA_PAD // 128
NC, NS, L = 2, 16, 16
NW = NC * NS
CH = 2048
KROW = CH // 128
NCHUNK = NE // CH
TMAX = -(-NCHUNK // NW)
APT = NA_PAD // NS


def _atom_body(charge_ref, idxm_ref, out_ref):
    charge = charge_ref[...]
    idxm = idxm_ref[...]
    vals = []
    for m in range(NM):
        mask = idxm == m
        qs = jnp.sum(jnp.where(mask, charge, 0.0))
        cnt = jnp.sum(jnp.where(mask, 1.0, 0.0))
        vals.append(-qs / jnp.maximum(cnt, 1.0))
    adj = jnp.zeros_like(charge)
    for m in range(NM):
        adj = jnp.where(idxm == m, vals[m], adj)
    out_ref[...] = charge + adj


def _stage1(charge2, idxm2):
    return pl.pallas_call(
        _atom_body,
        out_shape=jax.ShapeDtypeStruct((ROWS, 128), jnp.float32),
    )(charge2, idxm2)


def _rsqrt(x):
    i = plsc.bitcast(x, jnp.int32)
    i = 0x5F3759DF - lax.shift_right_logical(i, 1)
    y = plsc.bitcast(i, jnp.float32)
    for _ in range(3):
        y = y * (1.5 - 0.5 * x * y * y)
    return y


def _sc_body(qa_h, z_h, p0_h, rij_h, idxi_h, idxj_h, zero_h, out_hbm,
             qa_s, z_s, acc_s,
             idxi_v, idxj_v, rx_v, ry_v, rz_v,
             qi_v, qj_v, zi_v, zj_v, p0_v,
             epair_v, zbuf, zbufi, semL, semG):
    c = lax.axis_index("c")
    s = lax.axis_index("s")
    w = s * NC + c

    sl = pl.ds(s * APT, APT)
    for src_h, dst_s in ((qa_h, qa_s), (zero_h, acc_s)):
        pltpu.sync_copy(src_h.at[sl], zbuf)
        pltpu.sync_copy(zbuf, dst_s.at[sl])
    pltpu.sync_copy(z_h.at[sl], zbufi)
    pltpu.sync_copy(zbufi, z_s.at[sl])
    pltpu.sync_copy(p0_h, p0_v)
    plsc.subcore_barrier()

    def chunk_body(t, carry):
        cid = w + NW * t

        @pl.when(cid < NCHUNK)
        def _():
            dsc = [pltpu.async_copy(idxi_h.at[cid], idxi_v, semL),
                   pltpu.async_copy(idxj_h.at[cid], idxj_v, semL),
                   pltpu.async_copy(rij_h.at[0, cid], rx_v, semL),
                   pltpu.async_copy(rij_h.at[1, cid], ry_v, semL),
                   pltpu.async_copy(rij_h.at[2, cid], rz_v, semL)]
            for d in dsc:
                d.wait()

            gds = [pltpu.async_copy(qa_s.at[idxi_v], qi_v, semG),
                   pltpu.async_copy(qa_s.at[idxj_v], qj_v, semG),
                   pltpu.async_copy(z_s.at[idxi_v], zi_v, semG),
                   pltpu.async_copy(z_s.at[idxj_v], zj_v, semG)]
            for g in gds:
                g.wait()

            def vbody(j, vc):
                b = pl.ds(j * L, L)
                xv = rx_v[b]
                yv = ry_v[b]
                zv = rz_v[b]
                qi = qi_v[b]
                qj = qj_v[b]
                zi32 = zi_v[b]
                zj32 = zj_v[b]
                zi = zi32.astype(jnp.float32)
                zj = zj32.astype(jnp.float32)
                pi_ = plsc.load_gather(p0_v, [zi32])
                pj_ = plsc.load_gather(p0_v, [zj32])

                d2v = xv * xv + yv * yv + zv * zv
                rinv = _rsqrt(jnp.maximum(d2v, 1e-18))
                d = d2v * rinv
                dsafe = jnp.maximum(d, EPS)
                inv_d = 1.0 / dsafe
                incut = d < CUTOFF
                xc = d * (1.0 / CUTOFF)
                xc3 = xc * xc * xc
                fcut = jnp.where(
                    incut, 1.0 - xc3 * ((6.0 * xc - 15.0) * xc + 10.0), 0.0)
                x1 = jnp.clip((d - CUTON) * 0.2, 0.0, 1.0)
                sw = x1 * x1 * x1 * (x1 * (6.0 * x1 - 15.0) + 10.0)
                damped = _rsqrt(d2v + 1.0)
                coul = (1.0 - sw) * damped + sw * inv_d
                coul = jnp.where(incut, coul + d * 0.01 - 0.2, 0.0)
                ec = KEHALF * qi * qj * coul

                c6ij = ((0.4 * zi + 1.0) * (0.4 * zj + 1.0)
                        * jnp.exp(-0.25 * (qi + qj)))
                ds2 = dsafe * dsafe
                d6 = ds2 * ds2 * ds2
                r0 = 2.0 + 0.1 * (zi + zj)
                r02 = r0 * r0
                r06 = r02 * r02 * r02
                ea = (-0.5 * fcut) * c6ij / (d6 + r06)

                xs = dsafe * ((pi_ + pj_ + EPS) * INV_ADIV)
                phi = (0.18175 * jnp.exp(-3.19980 * xs)
                       + 0.50986 * jnp.exp(-0.94229 * xs)
                       + 0.28022 * jnp.exp(-0.40290 * xs)
                       + 0.02817 * jnp.exp(-0.20162 * xs))
                ez = KEHALF * zi * zj * inv_d * phi * fcut

                epair_v[b] = ec + ea + ez
                return vc

            lax.fori_loop(0, CH // L, vbody, 0)

            pltpu.sync_copy(epair_v, acc_s.at[idxi_v], add=True)

        return carry

    lax.fori_loop(0, TMAX, chunk_body, 0)

    plsc.subcore_barrier()
    pltpu.sync_copy(acc_s.at[pl.ds(s * APT, APT)], zbuf)
    pltpu.sync_copy(zbuf, out_hbm.at[c, pl.ds(s * APT, APT)])


_sc_call = functools.partial(
    pl.kernel,
    out_type=jax.ShapeDtypeStruct((NC, NA_PAD), jnp.float32),
    mesh=plsc.VectorSubcoreMesh(
        core_axis_name="c", subcore_axis_name="s",
        num_cores=NC, num_subcores=NS),
    compiler_params=pltpu.CompilerParams(needs_layout_passes=False),
    scratch_types=(
        [pltpu.VMEM_SHARED((NA_PAD,), jnp.float32),
         pltpu.VMEM_SHARED((NA_PAD,), jnp.int32),
         pltpu.VMEM_SHARED((NA_PAD,), jnp.float32)]
        + [pltpu.VMEM((CH,), jnp.int32) for _ in range(2)]
        + [pltpu.VMEM((CH,), jnp.float32) for _ in range(3)]
        + [pltpu.VMEM((CH,), jnp.float32) for _ in range(2)]
        + [pltpu.VMEM((CH,), jnp.int32) for _ in range(2)]
        + [pltpu.VMEM((128,), jnp.float32),
           pltpu.VMEM((CH,), jnp.float32),
           pltpu.VMEM((APT,), jnp.float32),
           pltpu.VMEM((APT,), jnp.int32),
           pltpu.SemaphoreType.DMA,
           pltpu.SemaphoreType.DMA]
    ),
)(_sc_body)


def kernel(yi, Z, Rij, idx_i, idx_j, idx_m):
    charge = yi[:, 1]
    pad = NA_PAD - NA
    charge2 = jnp.pad(charge, (0, pad)).reshape(ROWS, 128)
    idxm2 = jnp.pad(idx_m, (0, pad), constant_values=NM).reshape(ROWS, 128)
    qa_h = _stage1(charge2, idxm2).reshape(NA_PAD)
    z_h = jnp.pad(Z, (0, pad))
    p0tab = jnp.arange(128, dtype=jnp.float32) ** 0.23

    rijT = Rij.T.reshape(3, NCHUNK, CH)
    idxi3 = idx_i.reshape(NCHUNK, CH)
    idxj3 = idx_j.reshape(NCHUNK, CH)
    zero = jnp.zeros((NA_PAD,), jnp.float32)
    partials = _sc_call(qa_h, z_h, p0tab, rijT, idxi3, idxj3, zero)
    return (yi[:, 0] + partials[0, :NA] + partials[1, :NA])[:, None]


# speedup vs baseline: 460.9595x; 1.3352x over previous
import functools

import jax
import jax.numpy as jnp
from jax import lax
from jax.experimental import pallas as pl
from jax.experimental.pallas import tpu as pltpu
from jax.experimental.pallas import tpu_sc as plsc

KE = 14.399645351950548
KEHALF = KE / 2.0
CUTOFF = 10.0
CUTON = 2.5
EPS = 1e-9
A0 = 0.5291772105638411
INV_ADIV = 1.0 / (0.8854 * A0)

NA = 100000
NE = 6400000
NM = 64
NA_PAD = 102400
ROWS = NA_PAD // 128
NC, NS, L = 2, 16, 16
NW = NC * NS
CH = 2048
KROW = CH // 128
NCHUNK = NE // CH
TMAX = -(-NCHUNK // NW)
APT = NA_PAD // NS


def _atom_body(charge_ref, idxm_ref, out_ref):
    charge = charge_ref[...]
    idxm = idxm_ref[...]
    vals = []
    for m in range(NM):
        mask = idxm == m
        qs = jnp.sum(jnp.where(mask, charge, 0.0))
        cnt = jnp.sum(jnp.where(mask, 1.0, 0.0))
        vals.append(-qs / jnp.maximum(cnt, 1.0))
    adj = jnp.zeros_like(charge)
    for m in range(NM):
        adj = jnp.where(idxm == m, vals[m], adj)
    out_ref[...] = charge + adj


def _stage1(charge2, idxm2):
    return pl.pallas_call(
        _atom_body,
        out_shape=jax.ShapeDtypeStruct((ROWS, 128), jnp.float32),
    )(charge2, idxm2)


def _rsqrt(x):
    i = plsc.bitcast(x, jnp.int32)
    i = 0x5F3759DF - lax.shift_right_logical(i, 1)
    y = plsc.bitcast(i, jnp.float32)
    for _ in range(3):
        y = y * (1.5 - 0.5 * x * y * y)
    return y


def _sc_body(qa_h, z_h, p0_h, rij_h, idxi_h, idxj_h, zero_h, out_hbm,
             qa_s, z_s, acc_s, *rest):
    bufsA = rest[0:10]
    bufsB = rest[10:20]
    p0_v, zbuf, zbufi, semL, semG = rest[20:25]
    c = lax.axis_index("c")
    s = lax.axis_index("s")
    w = s * NC + c

    sl = pl.ds(s * APT, APT)
    for src_h, dst_s in ((qa_h, qa_s), (zero_h, acc_s)):
        pltpu.sync_copy(src_h.at[sl], zbuf)
        pltpu.sync_copy(zbuf, dst_s.at[sl])
    pltpu.sync_copy(z_h.at[sl], zbufi)
    pltpu.sync_copy(zbufi, z_s.at[sl])
    pltpu.sync_copy(p0_h, p0_v)
    plsc.subcore_barrier()

    def _linear_pairs(bufs, cid):
        idxi_v, idxj_v, rx_v, ry_v, rz_v = bufs[0:5]
        return ((idxi_h.at[cid], idxi_v), (idxj_h.at[cid], idxj_v),
                (rij_h.at[0, cid], rx_v), (rij_h.at[1, cid], ry_v),
                (rij_h.at[2, cid], rz_v))

    def _gather_pairs(bufs):
        idxi_v, idxj_v = bufs[0:2]
        qi_v, qj_v, zi_v, zj_v = bufs[5:9]
        return ((qa_s.at[idxi_v], qi_v), (qa_s.at[idxj_v], qj_v),
                (z_s.at[idxi_v], zi_v), (z_s.at[idxj_v], zj_v))

    def fire_linear(bufs, u):
        cid = w + NW * u

        @pl.when(cid < NCHUNK)
        def _():
            for src, dst in _linear_pairs(bufs, cid):
                pltpu.async_copy(src, dst, semL)

    def wait_linear(bufs, u):
        cid = w + NW * u

        @pl.when(cid < NCHUNK)
        def _():
            for src, dst in _linear_pairs(bufs, cid):
                pltpu.make_async_copy(src, dst, semL).wait()

    def fire_gathers(bufs, u):
        cid = w + NW * u

        @pl.when(cid < NCHUNK)
        def _():
            for src, dst in _gather_pairs(bufs):
                pltpu.async_copy(src, dst, semG)

    def wait_gathers(bufs, u):
        cid = w + NW * u

        @pl.when(cid < NCHUNK)
        def _():
            for src, dst in _gather_pairs(bufs):
                pltpu.make_async_copy(src, dst, semG).wait()

    def compute_scatter(bufs, u):
        cid = w + NW * u
        idxi_v, idxj_v, rx_v, ry_v, rz_v, qi_v, qj_v, zi_v, zj_v, epair_v = bufs

        @pl.when(cid < NCHUNK)
        def _():
            def vbody(j, vc):
                b = pl.ds(j * L, L)
                xv = rx_v[b]
                yv = ry_v[b]
                zv = rz_v[b]
                qi = qi_v[b]
                qj = qj_v[b]
                zi32 = zi_v[b]
                zj32 = zj_v[b]
                zi = zi32.astype(jnp.float32)
                zj = zj32.astype(jnp.float32)
                pi_ = plsc.load_gather(p0_v, [zi32])
                pj_ = plsc.load_gather(p0_v, [zj32])

                d2v = xv * xv + yv * yv + zv * zv
                rinv = _rsqrt(jnp.maximum(d2v, 1e-18))
                d = d2v * rinv
                dsafe = jnp.maximum(d, EPS)
                inv_d = 1.0 / dsafe
                incut = d < CUTOFF
                xc = d * (1.0 / CUTOFF)
                xc3 = xc * xc * xc
                fcut = jnp.where(
                    incut, 1.0 - xc3 * ((6.0 * xc - 15.0) * xc + 10.0), 0.0)
                x1 = jnp.clip((d - CUTON) * 0.2, 0.0, 1.0)
                sw = x1 * x1 * x1 * (x1 * (6.0 * x1 - 15.0) + 10.0)
                damped = _rsqrt(d2v + 1.0)
                coul = (1.0 - sw) * damped + sw * inv_d
                coul = jnp.where(incut, coul + d * 0.01 - 0.2, 0.0)
                ec = KEHALF * qi * qj * coul

                c6ij = ((0.4 * zi + 1.0) * (0.4 * zj + 1.0)
                        * jnp.exp(-0.25 * (qi + qj)))
                ds2 = dsafe * dsafe
                d6 = ds2 * ds2 * ds2
                r0 = 2.0 + 0.1 * (zi + zj)
                r02 = r0 * r0
                r06 = r02 * r02 * r02
                ea = (-0.5 * fcut) * c6ij / (d6 + r06)

                xs = dsafe * ((pi_ + pj_ + EPS) * INV_ADIV)
                phi = (0.18175 * jnp.exp(-3.19980 * xs)
                       + 0.50986 * jnp.exp(-0.94229 * xs)
                       + 0.28022 * jnp.exp(-0.40290 * xs)
                       + 0.02817 * jnp.exp(-0.20162 * xs))
                ez = KEHALF * zi * zj * inv_d * phi * fcut

                epair_v[b] = ec + ea + ez
                return vc

            lax.fori_loop(0, CH // L, vbody, 0)

            pltpu.sync_copy(epair_v, acc_s.at[idxi_v], add=True)

    fire_linear(bufsA, 0)
    wait_linear(bufsA, 0)
    fire_gathers(bufsA, 0)

    def pair_body(tt, carry):
        uA = 2 * tt
        uB = uA + 1
        uN = uA + 2
        fire_linear(bufsB, uB)
        wait_gathers(bufsA, uA)
        wait_linear(bufsB, uB)
        fire_gathers(bufsB, uB)
        compute_scatter(bufsA, uA)
        fire_linear(bufsA, uN)
        wait_gathers(bufsB, uB)
        wait_linear(bufsA, uN)
        fire_gathers(bufsA, uN)
        compute_scatter(bufsB, uB)
        return carry

    lax.fori_loop(0, TMAX // 2, pair_body, 0)

    plsc.subcore_barrier()
    pltpu.sync_copy(acc_s.at[pl.ds(s * APT, APT)], zbuf)
    pltpu.sync_copy(zbuf, out_hbm.at[c, pl.ds(s * APT, APT)])


_sc_call = functools.partial(
    pl.kernel,
    out_type=jax.ShapeDtypeStruct((NC, NA_PAD), jnp.float32),
    mesh=plsc.VectorSubcoreMesh(
        core_axis_name="c", subcore_axis_name="s",
        num_cores=NC, num_subcores=NS),
    compiler_params=pltpu.CompilerParams(needs_layout_passes=False),
    scratch_types=(
        [pltpu.VMEM_SHARED((NA_PAD,), jnp.float32),
         pltpu.VMEM_SHARED((NA_PAD,), jnp.int32),
         pltpu.VMEM_SHARED((NA_PAD,), jnp.float32)]
        + [pltpu.VMEM((CH,), jnp.int32),
           pltpu.VMEM((CH,), jnp.int32),
           pltpu.VMEM((CH,), jnp.float32),
           pltpu.VMEM((CH,), jnp.float32),
           pltpu.VMEM((CH,), jnp.float32),
           pltpu.VMEM((CH,), jnp.float32),
           pltpu.VMEM((CH,), jnp.float32),
           pltpu.VMEM((CH,), jnp.int32),
           pltpu.VMEM((CH,), jnp.int32),
           pltpu.VMEM((CH,), jnp.float32)] * 2
        + [pltpu.VMEM((128,), jnp.float32),
           pltpu.VMEM((APT,), jnp.float32),
           pltpu.VMEM((APT,), jnp.int32),
           pltpu.SemaphoreType.DMA,
           pltpu.SemaphoreType.DMA]
    ),
)(_sc_body)


def kernel(yi, Z, Rij, idx_i, idx_j, idx_m):
    charge = yi[:, 1]
    pad = NA_PAD - NA
    charge2 = jnp.pad(charge, (0, pad)).reshape(ROWS, 128)
    idxm2 = jnp.pad(idx_m, (0, pad), constant_values=NM).reshape(ROWS, 128)
    qa_h = _stage1(charge2, idxm2).reshape(NA_PAD)
    z_h = jnp.pad(Z, (0, pad))
    p0tab = jnp.arange(128, dtype=jnp.float32) ** 0.23

    rijT = Rij.T.reshape(3, NCHUNK, CH)
    idxi3 = idx_i.reshape(NCHUNK, CH)
    idxj3 = idx_j.reshape(NCHUNK, CH)
    zero = jnp.zeros((NA_PAD,), jnp.float32)
    partials = _sc_call(qa_h, z_h, p0tab, rijT, idxi3, idxj3, zero)
    return (yi[:, 0] + partials[0, :NA] + partials[1, :NA])[:, None]
